# baseline (speedup 1.0000x reference)
import jax
import jax.numpy as jnp
from jax.experimental import pallas as pl


def _copy_kernel(probs_ref, out_ref):
    out_ref[...] = probs_ref[...]


def kernel(x, top_k_probs, top_k_indices, router_logits, w_gate, w_noise):
    return pl.pallas_call(
        _copy_kernel,
        out_shape=jax.ShapeDtypeStruct(top_k_probs.shape, top_k_probs.dtype),
    )(top_k_probs)
